# Initial kernel scaffold; baseline (speedup 1.0000x reference)
import jax
import jax.numpy as jnp
from jax.experimental import pallas as pl


def kernel(deltas, class_logits, anchors):
    raise NotImplementedError("write your pallas kernel here")


# speedup vs baseline: 4.5598x; 4.5598x over previous
import functools

import jax
import jax.numpy as jnp
from jax import lax
from jax.experimental import pallas as pl
from jax.experimental.pallas import tpu as pltpu

BATCH = 2
N = 20000
OUT_NUM = 2000
IOU_T = 0.7
SCORE_T = 0.05
NEG = -1e10

LANES = 128
ROWS = 160
NP = ROWS * LANES


def _nms_body(pack_ref, out_ref):
    arr = pack_ref[0]
    dy = arr[0]
    dx = arr[1]
    eh = arr[2]
    ew = arr[3]
    a0 = arr[4]
    a1 = arr[5]
    a2 = arr[6]
    a3 = arr[7]
    l0 = arr[8]
    l1 = arr[9]
    fg = arr[10]

    h = a2 - a0
    w = a3 - a1
    cy = (a2 + a0) * 0.5
    cx = (a3 + a1) * 0.5
    cy = cy + dy * h
    cx = cx + dx * w
    hh = h * eh
    ww = w * ew
    y1 = cy - hh * 0.5
    x1 = cx - ww * 0.5
    y2 = cy + hh * 0.5
    x2 = cx + ww * 0.5

    ymin = jnp.minimum(y1, y2)
    ymax = jnp.maximum(y1, y2)
    xmin = jnp.minimum(x1, x2)
    xmax = jnp.maximum(x1, x2)
    area = (ymax - ymin) * (xmax - xmin)

    flat = (lax.broadcasted_iota(jnp.int32, (ROWS, LANES), 0) * LANES
            + lax.broadcasted_iota(jnp.int32, (ROWS, LANES), 1))
    in_range = flat < N
    sm0 = jnp.where(jnp.logical_and(in_range, fg > SCORE_T), fg, NEG)

    li = lax.broadcasted_iota(jnp.int32, (1, LANES), 1)
    big = jnp.int32(2**30)

    def body(i, sm):
        m = jnp.max(sm)
        idx = jnp.min(jnp.where(sm == m, flat, big))
        valid = m > -5e9
        onehot = flat == idx
        vmf = jnp.where(valid, jnp.float32(1.0), jnp.float32(0.0))

        ys1 = jnp.sum(jnp.where(onehot, y1, 0.0))
        xs1 = jnp.sum(jnp.where(onehot, x1, 0.0))
        ys2 = jnp.sum(jnp.where(onehot, y2, 0.0))
        xs2 = jnp.sum(jnp.where(onehot, x2, 0.0))
        sl0 = jnp.sum(jnp.where(onehot, l0, 0.0))
        sl1 = jnp.sum(jnp.where(onehot, l1, 0.0))

        ymin1 = jnp.minimum(ys1, ys2)
        ymax1 = jnp.maximum(ys1, ys2)
        xmin1 = jnp.minimum(xs1, xs2)
        xmax1 = jnp.maximum(xs1, xs2)
        ih = jnp.maximum(0.0, jnp.minimum(ymax1, ymax) - jnp.maximum(ymin1, ymin))
        iw = jnp.maximum(0.0, jnp.minimum(xmax1, xmax) - jnp.maximum(xmin1, xmin))
        inter = ih * iw
        area1 = (ymax1 - ymin1) * (xmax1 - xmin1)
        union = area1 + area - inter
        upos = union > 0
        iou = jnp.where(upos, inter / jnp.where(upos, union, 1.0), 0.0)

        sm = jnp.where(iou > IOU_T, NEG, sm)
        sm = jnp.where(onehot, NEG, sm)

        row = jnp.where(li == 0, ys1,
              jnp.where(li == 1, xs1,
              jnp.where(li == 2, ys2,
              jnp.where(li == 3, xs2,
              jnp.where(li == 5, m,
              jnp.where(li == 7, sl0,
              jnp.where(li == 8, sl1,
              jnp.where(jnp.logical_or(li == 4,
                        jnp.logical_or(li == 6, li == 9)),
                        jnp.float32(1.0), jnp.float32(0.0))))))))) * vmf
        out_ref[0, pl.ds(i, 1), :] = row.astype(jnp.float32)
        return sm

    lax.fori_loop(0, OUT_NUM, body, sm0)


@functools.partial(jax.jit, static_argnames=())
def kernel(deltas, class_logits, anchors):
    class_scores = jax.nn.softmax(class_logits, axis=-1)
    fg = jnp.max(class_scores[..., 1:], axis=-1)
    scaled = deltas * jnp.array([0.1, 0.1, 0.2, 0.2], dtype=jnp.float32)
    dy = scaled[..., 0]
    dx = scaled[..., 1]
    eh = jnp.exp(scaled[..., 2])
    ew = jnp.exp(scaled[..., 3])
    a0 = anchors[..., 0]
    a1 = anchors[..., 1]
    a2 = anchors[..., 2]
    a3 = anchors[..., 3]
    l0 = class_logits[..., 0]
    l1 = class_logits[..., 1]

    def prep(x):
        return jnp.pad(x, ((0, 0), (0, NP - N))).reshape(BATCH, ROWS, LANES)

    pack = jnp.stack(
        [prep(x) for x in (dy, dx, eh, ew, a0, a1, a2, a3, l0, l1, fg)], axis=1)

    out = pl.pallas_call(
        _nms_body,
        grid=(BATCH,),
        in_specs=[pl.BlockSpec((1, 11, ROWS, LANES), lambda b: (b, 0, 0, 0))],
        out_specs=pl.BlockSpec((1, OUT_NUM, LANES), lambda b: (b, 0, 0)),
        out_shape=jax.ShapeDtypeStruct((BATCH, OUT_NUM, LANES), jnp.float32),
        compiler_params=pltpu.CompilerParams(
            dimension_semantics=("parallel",)),
    )(pack)

    return (out[..., 0:5], out[..., 5:7], out[..., 7:10])
